# baseline (speedup 1.0000x reference)
import functools

import jax
import jax.numpy as jnp
from jax import lax
from jax.experimental import pallas as pl
from jax.experimental.pallas import tpu as pltpu
from jax.experimental.pallas import tpu_sc as plsc

N = 10000
N_PAD = 10240
PAD_NODE = 10200
EM_PAD = 163840
EW_PAD = 81920
LAT = 128
BLK = 512
NC = 2
NS = 16
NW = NC * NS

def _dot(a, b):
    return jnp.dot(a, b, preferred_element_type=jnp.float32)


def _ln(h, g, b):
    mu = jnp.mean(h, axis=-1, keepdims=True)
    c = h - mu
    var = jnp.mean(c * c, axis=-1, keepdims=True)
    return c / jnp.sqrt(var + 1e-5) * g + b


def _full(shape):
    return pl.BlockSpec(shape, lambda i: (0,) * len(shape))


def _mlp_ln_body(x_ref, w1_ref, b1_ref, w2_ref, b2_ref, g_ref, be_ref, o_ref):
    h = jnp.maximum(_dot(x_ref[...], w1_ref[...]) + b1_ref[...], 0.0)
    o = _dot(h, w2_ref[...]) + b2_ref[...]
    o_ref[...] = _ln(o, g_ref[...], be_ref[...])


def _mlp_ln(x, p, blk, r_out):
    r, din = x.shape
    dout = p['W2'].shape[1]
    return pl.pallas_call(
        _mlp_ln_body,
        grid=(r // blk,),
        in_specs=[
            pl.BlockSpec((blk, din), lambda i: (i, 0)),
            _full((din, p['W1'].shape[1])),
            _full((1, p['W1'].shape[1])),
            _full((p['W2'].shape[0], dout)),
            _full((1, dout)),
            _full((1, dout)),
            _full((1, dout)),
        ],
        out_specs=pl.BlockSpec((blk, dout), lambda i: (i, 0)),
        out_shape=jax.ShapeDtypeStruct((r_out, dout), jnp.float32),
    )(x, p['W1'], p['b1'].reshape(1, -1), p['W2'], p['b2'].reshape(1, -1),
      p['g'].reshape(1, -1), p['be'].reshape(1, -1))


def _edge_body(em_ref, gs_ref, gd_ref, w1a_ref, w1b_ref, w1c_ref, b1_ref,
               w2_ref, b2_ref, ga_ref, be_ref, eu_ref, en_ref):
    x = (_dot(em_ref[...], w1a_ref[...]) + _dot(gs_ref[...], w1b_ref[...])
         + _dot(gd_ref[...], w1c_ref[...]) + b1_ref[...])
    h = jnp.maximum(x, 0.0)
    o = _ln(_dot(h, w2_ref[...]) + b2_ref[...], ga_ref[...], be_ref[...])
    eu_ref[...] = o
    en_ref[...] = em_ref[...] + o


def _edge_last_body(em_ref, gs_ref, gd_ref, w1a_ref, w1b_ref, w1c_ref,
                    b1_ref, w2_ref, b2_ref, ga_ref, be_ref, eu_ref):
    x = (_dot(em_ref[...], w1a_ref[...]) + _dot(gs_ref[...], w1b_ref[...])
         + _dot(gd_ref[...], w1c_ref[...]) + b1_ref[...])
    h = jnp.maximum(x, 0.0)
    eu_ref[...] = _ln(_dot(h, w2_ref[...]) + b2_ref[...], ga_ref[...],
                      be_ref[...])


def _edge_update_last(em, gs, gd, p, blk=2048):
    r = em.shape[0]
    w1 = p['W1']
    spec = pl.BlockSpec((blk, LAT), lambda i: (i, 0))
    return pl.pallas_call(
        _edge_last_body,
        grid=(r // blk,),
        in_specs=[spec, spec, spec, _full((LAT, LAT)), _full((LAT, LAT)),
                  _full((LAT, LAT)), _full((1, LAT)), _full((LAT, LAT)),
                  _full((1, LAT)), _full((1, LAT)), _full((1, LAT))],
        out_specs=spec,
        out_shape=jax.ShapeDtypeStruct((r, LAT), jnp.float32),
    )(em, gs, gd, w1[0:128], w1[128:256], w1[256:384], p['b1'].reshape(1, -1),
      p['W2'], p['b2'].reshape(1, -1), p['g'].reshape(1, -1),
      p['be'].reshape(1, -1))


def _edge_update(em, gs, gd, p, blk=2048):
    r = em.shape[0]
    w1 = p['W1']
    spec = pl.BlockSpec((blk, LAT), lambda i: (i, 0))
    shp = jax.ShapeDtypeStruct((r, LAT), jnp.float32)
    return pl.pallas_call(
        _edge_body,
        grid=(r // blk,),
        in_specs=[spec, spec, spec, _full((LAT, LAT)), _full((LAT, LAT)),
                  _full((LAT, LAT)), _full((1, LAT)), _full((LAT, LAT)),
                  _full((1, LAT)), _full((1, LAT)), _full((1, LAT))],
        out_specs=[spec, spec],
        out_shape=[shp, shp],
    )(em, gs, gd, w1[0:128], w1[128:256], w1[256:384], p['b1'].reshape(1, -1),
      p['W2'], p['b2'].reshape(1, -1), p['g'].reshape(1, -1),
      p['be'].reshape(1, -1))


def _node_body(v_ref, am_ref, aw_ref, w1v_ref, w1m_ref, w1w_ref, b1_ref,
               w2_ref, b2_ref, ga_ref, be_ref, o_ref):
    x = (_dot(v_ref[...], w1v_ref[...])
         + _dot(am_ref[0] + am_ref[1], w1m_ref[...])
         + _dot(aw_ref[0] + aw_ref[1], w1w_ref[...])
         + b1_ref[...])
    h = jnp.maximum(x, 0.0)
    o = _ln(_dot(h, w2_ref[...]) + b2_ref[...], ga_ref[...], be_ref[...])
    o_ref[...] = v_ref[...] + o


def _node_update(v, am, aw, p):
    r = v.shape[0]
    spec = pl.BlockSpec((BLK, LAT), lambda i: (i, 0))
    spec2 = pl.BlockSpec((NC, BLK, LAT), lambda i: (0, i, 0))
    w1 = p['W1']
    return pl.pallas_call(
        _node_body,
        grid=(r // BLK,),
        in_specs=[spec, spec2, spec2, _full((LAT, LAT)), _full((LAT, LAT)),
                  _full((LAT, LAT)), _full((1, LAT)), _full((LAT, LAT)),
                  _full((1, LAT)), _full((1, LAT)), _full((1, LAT))],
        out_specs=spec,
        out_shape=jax.ShapeDtypeStruct((r, LAT), jnp.float32),
    )(v, am, aw, w1[0:128], w1[128:256], w1[256:384], p['b1'].reshape(1, -1),
      p['W2'], p['b2'].reshape(1, -1), p['g'].reshape(1, -1),
      p['be'].reshape(1, -1))


def _dec_body(v_ref, w1_ref, b1_ref, w2_ref, b2_ref, o_ref):
    h = jnp.maximum(_dot(v_ref[...], w1_ref[...]) + b1_ref[...], 0.0)
    o_ref[...] = _dot(h, w2_ref[...]) + b2_ref[...]


def _decode(v, w1, b1, w2, b2):
    r = v.shape[0]
    spec = pl.BlockSpec((BLK, LAT), lambda i: (i, 0))
    return pl.pallas_call(
        _dec_body,
        grid=(r // BLK,),
        in_specs=[spec, _full((LAT, LAT)), _full((1, LAT)), _full((LAT, LAT)),
                  _full((1, LAT))],
        out_specs=spec,
        out_shape=jax.ShapeDtypeStruct((r, LAT), jnp.float32),
    )(v, w1, b1.reshape(1, -1), w2, b2.reshape(1, -1))


def _make_gather(e_pad):
    cpw = e_pad // NW
    nch = cpw // 128
    mesh = plsc.VectorSubcoreMesh(core_axis_name="c", subcore_axis_name="s")
    shp = jax.ShapeDtypeStruct((e_pad, LAT), jnp.float32)

    rpsv = N_PAD // NS

    @functools.partial(
        pl.kernel, mesh=mesh,
        out_type=[shp, shp],
        scratch_types=[
            pltpu.VMEM((nch, 128), jnp.int32),
            pltpu.VMEM((nch, 128), jnp.int32),
            pltpu.VMEM((2, 64, LAT), jnp.float32),
            pltpu.VMEM((2, 64, LAT), jnp.float32),
            pltpu.VMEM_SHARED((N_PAD, LAT), jnp.float32),
            pltpu.SemaphoreType.DMA((2,)),
            pltpu.SemaphoreType.DMA((2,)),
            pltpu.SemaphoreType.DMA((2,)),
            pltpu.SemaphoreType.DMA((2,)),
        ])
    def k(v_hbm, src_hbm, dst_hbm, gs_hbm, gd_hbm, si_v, di_v, ba, bb,
          vsh, ga, gb, wa, wb):
        w = lax.axis_index("c") * NS + lax.axis_index("s")
        s0 = lax.axis_index("s")
        pltpu.sync_copy(v_hbm.at[pl.ds(s0 * rpsv, rpsv)],
                        vsh.at[pl.ds(s0 * rpsv, rpsv)])
        pltpu.sync_copy(src_hbm.at[w], si_v)
        pltpu.sync_copy(dst_hbm.at[w], di_v)
        plsc.subcore_barrier()

        def start_g(j, h):
            idx_s = si_v.at[j, pl.ds(h * 64, 64)]
            idx_d = di_v.at[j, pl.ds(h * 64, 64)]
            pltpu.async_copy(vsh.at[idx_s], ba.at[h], ga.at[h])
            pltpu.async_copy(vsh.at[idx_d], bb.at[h], gb.at[h])

        def wait_g(h):
            i0 = si_v.at[0, pl.ds(0, 64)]
            pltpu.make_async_copy(vsh.at[i0], ba.at[h], ga.at[h]).wait()
            pltpu.make_async_copy(vsh.at[i0], bb.at[h], gb.at[h]).wait()

        def start_w(j, h):
            r = pl.ds((w * nch + j) * 128 + h * 64, 64)
            pltpu.async_copy(ba.at[h], gs_hbm.at[r], wa.at[h])
            pltpu.async_copy(bb.at[h], gd_hbm.at[r], wb.at[h])

        def wait_w(h):
            r0 = pl.ds(w * nch * 128, 64)
            pltpu.make_async_copy(ba.at[h], gs_hbm.at[r0], wa.at[h]).wait()
            pltpu.make_async_copy(bb.at[h], gd_hbm.at[r0], wb.at[h]).wait()

        start_g(0, 0)

        def body(j, carry):
            @pl.when(j > 0)
            def _():
                wait_w(1)
            start_g(j, 1)
            wait_g(0)
            start_w(j, 0)
            wait_w(0)

            @pl.when(j + 1 < nch)
            def _():
                start_g(j + 1, 0)
            wait_g(1)
            start_w(j, 1)
            return carry
        lax.fori_loop(0, nch, body, 0)
        wait_w(1)

    return k


def _make_scatter(e_pad):
    cpw = e_pad // NW
    nch = cpw // 128
    rps = N_PAD // NS
    mesh = plsc.VectorSubcoreMesh(core_axis_name="c", subcore_axis_name="s")

    @functools.partial(
        pl.kernel, mesh=mesh,
        out_type=jax.ShapeDtypeStruct((NC, N_PAD, LAT), jnp.float32),
        scratch_types=[
            pltpu.VMEM((nch, 128), jnp.int32),
            pltpu.VMEM((2, 128, LAT), jnp.float32),
            pltpu.VMEM_SHARED((N_PAD, LAT), jnp.float32),
            pltpu.SemaphoreType.DMA((2,)),
            pltpu.SemaphoreType.DMA((2,)),
            pltpu.SemaphoreType.DMA,
        ])
    def k(eu_hbm, dst_hbm, out_hbm, di_v, buf, acc, rs, ss, ds):
        c0 = lax.axis_index("c")
        s0 = lax.axis_index("s")
        w = c0 * NS + s0

        def zrow(r, c2):
            for cc in range(8):
                buf[0, r, pl.ds(cc * 16, 16)] = jnp.zeros((16,), jnp.float32)
            return c2
        lax.fori_loop(0, 128, zrow, 0, unroll=2)
        for t in range(rps // 128):
            pltpu.sync_copy(buf.at[0], acc.at[pl.ds(s0 * rps + t * 128, 128)])
        plsc.subcore_barrier()

        pltpu.sync_copy(dst_hbm.at[w], di_v)

        def start_r(cn, p):
            pltpu.async_copy(eu_hbm.at[pl.ds((w * nch + cn) * 128, 128)],
                             buf.at[p], rs.at[p])

        def wait_r(p):
            pltpu.make_async_copy(eu_hbm.at[pl.ds(w * nch * 128, 128)],
                                  buf.at[p], rs.at[p]).wait()

        def start_s(cn, p):
            pltpu.async_copy(buf.at[p], acc.at[di_v.at[cn]], ss.at[p], add=True)

        def wait_s(p):
            pltpu.make_async_copy(buf.at[p], acc.at[di_v.at[0]], ss.at[p]).wait()

        start_r(0, 0)

        def body(j2, carry):
            cn = j2 * 2

            @pl.when(j2 > 0)
            def _():
                wait_s(1)
            start_r(cn + 1, 1)
            wait_r(0)
            start_s(cn, 0)
            wait_s(0)

            @pl.when(cn + 2 < nch)
            def _():
                start_r(cn + 2, 0)
            wait_r(1)
            start_s(cn + 1, 1)
            return carry
        lax.fori_loop(0, nch // 2, body, 0)
        wait_s(1)
        plsc.subcore_barrier()

        for t in range(rps // 128):
            r0 = s0 * rps + t * 128
            pltpu.sync_copy(acc.at[pl.ds(r0, 128)], out_hbm.at[c0, pl.ds(r0, 128)])

    return k


def kernel(node_features, mesh_edge_features, world_edge_features,
           mesh_edge_index, world_edge_index, params):
    f32 = jnp.float32
    em_e = mesh_edge_features.shape[0]
    ew_e = world_edge_features.shape[0]
    msrc = jnp.pad(mesh_edge_index[0].astype(jnp.int32),
                   (0, EM_PAD - em_e)).reshape(NW, -1, 128)
    mdst = jnp.pad(mesh_edge_index[1].astype(jnp.int32), (0, EM_PAD - em_e),
                   constant_values=PAD_NODE).reshape(NW, -1, 128)
    wsrc = jnp.pad(world_edge_index[0].astype(jnp.int32),
                   (0, EW_PAD - ew_e)).reshape(NW, -1, 128)
    wdst = jnp.pad(world_edge_index[1].astype(jnp.int32), (0, EW_PAD - ew_e),
                   constant_values=PAD_NODE).reshape(NW, -1, 128)

    v = _mlp_ln(node_features.astype(f32), params['enc_node'], 400, N_PAD)
    em = _mlp_ln(mesh_edge_features.astype(f32), params['enc_mesh'], 640,
                 EM_PAD)
    ew = _mlp_ln(world_edge_features.astype(f32), params['enc_world'], 640,
                 EW_PAD)

    gather_m = _make_gather(EM_PAD)
    gather_w = _make_gather(EW_PAD)
    scatter_m = _make_scatter(EM_PAD)
    scatter_w = _make_scatter(EW_PAD)

    n_steps = len(params['proc'])
    for si, ps in enumerate(params['proc']):
        last = si == n_steps - 1
        gs_m, gd_m = gather_m(v, msrc, mdst)
        gs_w, gd_w = gather_w(v, wsrc, wdst)
        if last:
            em_u = _edge_update_last(em, gs_m, gd_m, ps['mesh_edge'])
            ew_u = _edge_update_last(ew, gs_w, gd_w, ps['world_edge'])
        else:
            em_u, em = _edge_update(em, gs_m, gd_m, ps['mesh_edge'])
            ew_u, ew = _edge_update(ew, gs_w, gd_w, ps['world_edge'])
        am = scatter_m(em_u, mdst)
        aw = scatter_w(ew_u, wdst)
        v = _node_update(v, am, aw, ps['node'])

    dec = params['decoder']
    w2d = jnp.pad(dec['W2'], ((0, 0), (0, LAT - dec['W2'].shape[1])))
    b2d = jnp.pad(dec['b2'], (0, LAT - dec['b2'].shape[0]))
    out = _decode(v, dec['W1'], dec['b1'], w2d, b2d)
    return out[:N, :dec['W2'].shape[1]]


# speedup vs baseline: 1.1715x; 1.1715x over previous
import functools

import jax
import jax.numpy as jnp
from jax import lax
from jax.experimental import pallas as pl
from jax.experimental.pallas import tpu as pltpu
from jax.experimental.pallas import tpu_sc as plsc

N = 10000
N_PAD = 10240
PAD_NODE = 10200
EM_PAD = 163840
EW_PAD = 81920
LAT = 128
BLK = 512
NC = 2
NS = 16
NW = NC * NS

def _dot(a, b):
    return jnp.dot(a, b, preferred_element_type=jnp.float32)


def _ln(h, g, b):
    mu = jnp.mean(h, axis=-1, keepdims=True)
    c = h - mu
    var = jnp.mean(c * c, axis=-1, keepdims=True)
    return c / jnp.sqrt(var + 1e-5) * g + b


def _full(shape):
    return pl.BlockSpec(shape, lambda i: (0,) * len(shape))


def _mlp_ln_body(x_ref, w1_ref, b1_ref, w2_ref, b2_ref, g_ref, be_ref, o_ref):
    h = jnp.maximum(_dot(x_ref[...], w1_ref[...]) + b1_ref[...], 0.0)
    o = _dot(h, w2_ref[...]) + b2_ref[...]
    o_ref[...] = _ln(o, g_ref[...], be_ref[...])


def _mlp_ln(x, p, blk, r_out):
    r, din = x.shape
    dout = p['W2'].shape[1]
    return pl.pallas_call(
        _mlp_ln_body,
        grid=(r // blk,),
        in_specs=[
            pl.BlockSpec((blk, din), lambda i: (i, 0)),
            _full((din, p['W1'].shape[1])),
            _full((1, p['W1'].shape[1])),
            _full((p['W2'].shape[0], dout)),
            _full((1, dout)),
            _full((1, dout)),
            _full((1, dout)),
        ],
        out_specs=pl.BlockSpec((blk, dout), lambda i: (i, 0)),
        out_shape=jax.ShapeDtypeStruct((r_out, dout), jnp.float32),
    )(x, p['W1'], p['b1'].reshape(1, -1), p['W2'], p['b2'].reshape(1, -1),
      p['g'].reshape(1, -1), p['be'].reshape(1, -1))


def _enc_packed_body(x_ref, w1r_ref, b1r_ref, w2_ref, b2_ref, g_ref, be_ref,
                     o_ref):
    h = jnp.maximum(_dot(x_ref[...], w1r_ref[...]) + b1r_ref[...], 0.0)
    blk = h.shape[0]
    h = h.reshape(blk * 8, LAT)
    o = _dot(h, w2_ref[...]) + b2_ref[...]
    o_ref[...] = _ln(o, g_ref[...], be_ref[...])


def _enc_packed(x, p, blk, r_out):
    rp = x.shape[0]
    w1, b1 = p['W1'], p['b1']
    w1r = jnp.zeros((128, 8 * LAT), jnp.float32)
    for j in range(8):
        w1r = w1r.at[16 * j:16 * (j + 1), LAT * j:LAT * (j + 1)].set(w1)
    b1r = jnp.tile(b1, 8).reshape(1, 8 * LAT)
    return pl.pallas_call(
        _enc_packed_body,
        grid=(rp // blk,),
        in_specs=[
            pl.BlockSpec((blk, 128), lambda i: (i, 0)),
            _full((128, 8 * LAT)),
            _full((1, 8 * LAT)),
            _full((LAT, LAT)),
            _full((1, LAT)),
            _full((1, LAT)),
            _full((1, LAT)),
        ],
        out_specs=pl.BlockSpec((blk * 8, LAT), lambda i: (i, 0)),
        out_shape=jax.ShapeDtypeStruct((r_out, LAT), jnp.float32),
    )(x, w1r, b1r, p['W2'], p['b2'].reshape(1, -1), p['g'].reshape(1, -1),
      p['be'].reshape(1, -1))


def _edge_body(em_ref, gs_ref, gd_ref, w1a_ref, w1b_ref, w1c_ref, b1_ref,
               w2_ref, b2_ref, ga_ref, be_ref, eu_ref, en_ref):
    x = (_dot(em_ref[...], w1a_ref[...]) + _dot(gs_ref[...], w1b_ref[...])
         + _dot(gd_ref[...], w1c_ref[...]) + b1_ref[...])
    h = jnp.maximum(x, 0.0)
    o = _ln(_dot(h, w2_ref[...]) + b2_ref[...], ga_ref[...], be_ref[...])
    eu_ref[...] = o
    en_ref[...] = em_ref[...] + o


def _edge_last_body(em_ref, gs_ref, gd_ref, w1a_ref, w1b_ref, w1c_ref,
                    b1_ref, w2_ref, b2_ref, ga_ref, be_ref, eu_ref):
    x = (_dot(em_ref[...], w1a_ref[...]) + _dot(gs_ref[...], w1b_ref[...])
         + _dot(gd_ref[...], w1c_ref[...]) + b1_ref[...])
    h = jnp.maximum(x, 0.0)
    eu_ref[...] = _ln(_dot(h, w2_ref[...]) + b2_ref[...], ga_ref[...],
                      be_ref[...])


def _edge_update_last(em, gs, gd, p, blk=2048):
    r = em.shape[0]
    w1 = p['W1']
    spec = pl.BlockSpec((blk, LAT), lambda i: (i, 0))
    return pl.pallas_call(
        _edge_last_body,
        grid=(r // blk,),
        in_specs=[spec, spec, spec, _full((LAT, LAT)), _full((LAT, LAT)),
                  _full((LAT, LAT)), _full((1, LAT)), _full((LAT, LAT)),
                  _full((1, LAT)), _full((1, LAT)), _full((1, LAT))],
        out_specs=spec,
        out_shape=jax.ShapeDtypeStruct((r, LAT), jnp.float32),
    )(em, gs, gd, w1[0:128], w1[128:256], w1[256:384], p['b1'].reshape(1, -1),
      p['W2'], p['b2'].reshape(1, -1), p['g'].reshape(1, -1),
      p['be'].reshape(1, -1))


def _edge_update(em, gs, gd, p, blk=2048):
    r = em.shape[0]
    w1 = p['W1']
    spec = pl.BlockSpec((blk, LAT), lambda i: (i, 0))
    shp = jax.ShapeDtypeStruct((r, LAT), jnp.float32)
    return pl.pallas_call(
        _edge_body,
        grid=(r // blk,),
        in_specs=[spec, spec, spec, _full((LAT, LAT)), _full((LAT, LAT)),
                  _full((LAT, LAT)), _full((1, LAT)), _full((LAT, LAT)),
                  _full((1, LAT)), _full((1, LAT)), _full((1, LAT))],
        out_specs=[spec, spec],
        out_shape=[shp, shp],
    )(em, gs, gd, w1[0:128], w1[128:256], w1[256:384], p['b1'].reshape(1, -1),
      p['W2'], p['b2'].reshape(1, -1), p['g'].reshape(1, -1),
      p['be'].reshape(1, -1))


def _node_body(v_ref, am_ref, aw_ref, w1v_ref, w1m_ref, w1w_ref, b1_ref,
               w2_ref, b2_ref, ga_ref, be_ref, o_ref):
    x = (_dot(v_ref[...], w1v_ref[...])
         + _dot(am_ref[0] + am_ref[1], w1m_ref[...])
         + _dot(aw_ref[0] + aw_ref[1], w1w_ref[...])
         + b1_ref[...])
    h = jnp.maximum(x, 0.0)
    o = _ln(_dot(h, w2_ref[...]) + b2_ref[...], ga_ref[...], be_ref[...])
    o_ref[...] = v_ref[...] + o


def _node_update(v, am, aw, p):
    r = v.shape[0]
    spec = pl.BlockSpec((BLK, LAT), lambda i: (i, 0))
    spec2 = pl.BlockSpec((NC, BLK, LAT), lambda i: (0, i, 0))
    w1 = p['W1']
    return pl.pallas_call(
        _node_body,
        grid=(r // BLK,),
        in_specs=[spec, spec2, spec2, _full((LAT, LAT)), _full((LAT, LAT)),
                  _full((LAT, LAT)), _full((1, LAT)), _full((LAT, LAT)),
                  _full((1, LAT)), _full((1, LAT)), _full((1, LAT))],
        out_specs=spec,
        out_shape=jax.ShapeDtypeStruct((r, LAT), jnp.float32),
    )(v, am, aw, w1[0:128], w1[128:256], w1[256:384], p['b1'].reshape(1, -1),
      p['W2'], p['b2'].reshape(1, -1), p['g'].reshape(1, -1),
      p['be'].reshape(1, -1))


def _dec_body(v_ref, w1_ref, b1_ref, w2_ref, b2_ref, o_ref):
    h = jnp.maximum(_dot(v_ref[...], w1_ref[...]) + b1_ref[...], 0.0)
    o_ref[...] = _dot(h, w2_ref[...]) + b2_ref[...]


def _decode(v, w1, b1, w2, b2):
    r = v.shape[0]
    spec = pl.BlockSpec((BLK, LAT), lambda i: (i, 0))
    return pl.pallas_call(
        _dec_body,
        grid=(r // BLK,),
        in_specs=[spec, _full((LAT, LAT)), _full((1, LAT)), _full((LAT, LAT)),
                  _full((1, LAT))],
        out_specs=spec,
        out_shape=jax.ShapeDtypeStruct((r, LAT), jnp.float32),
    )(v, w1, b1.reshape(1, -1), w2, b2.reshape(1, -1))


def _make_gather(e_pad):
    cpw = e_pad // NW
    nch = cpw // 128
    mesh = plsc.VectorSubcoreMesh(core_axis_name="c", subcore_axis_name="s")
    shp = jax.ShapeDtypeStruct((e_pad, LAT), jnp.float32)

    rpsv = N_PAD // NS

    @functools.partial(
        pl.kernel, mesh=mesh,
        out_type=[shp, shp],
        scratch_types=[
            pltpu.VMEM((nch, 128), jnp.int32),
            pltpu.VMEM((nch, 128), jnp.int32),
            pltpu.VMEM((2, 64, LAT), jnp.float32),
            pltpu.VMEM((2, 64, LAT), jnp.float32),
            pltpu.VMEM_SHARED((N_PAD, LAT), jnp.float32),
            pltpu.SemaphoreType.DMA((2,)),
            pltpu.SemaphoreType.DMA((2,)),
            pltpu.SemaphoreType.DMA((2,)),
            pltpu.SemaphoreType.DMA((2,)),
        ])
    def k(v_hbm, src_hbm, dst_hbm, gs_hbm, gd_hbm, si_v, di_v, ba, bb,
          vsh, ga, gb, wa, wb):
        w = lax.axis_index("c") * NS + lax.axis_index("s")
        s0 = lax.axis_index("s")
        pltpu.sync_copy(v_hbm.at[pl.ds(s0 * rpsv, rpsv)],
                        vsh.at[pl.ds(s0 * rpsv, rpsv)])
        pltpu.sync_copy(src_hbm.at[w], si_v)
        pltpu.sync_copy(dst_hbm.at[w], di_v)
        plsc.subcore_barrier()

        def start_g(j, h):
            idx_s = si_v.at[j, pl.ds(h * 64, 64)]
            idx_d = di_v.at[j, pl.ds(h * 64, 64)]
            pltpu.async_copy(vsh.at[idx_s], ba.at[h], ga.at[h])
            pltpu.async_copy(vsh.at[idx_d], bb.at[h], gb.at[h])

        def wait_g(h):
            i0 = si_v.at[0, pl.ds(0, 64)]
            pltpu.make_async_copy(vsh.at[i0], ba.at[h], ga.at[h]).wait()
            pltpu.make_async_copy(vsh.at[i0], bb.at[h], gb.at[h]).wait()

        def start_w(j, h):
            r = pl.ds((w * nch + j) * 128 + h * 64, 64)
            pltpu.async_copy(ba.at[h], gs_hbm.at[r], wa.at[h])
            pltpu.async_copy(bb.at[h], gd_hbm.at[r], wb.at[h])

        def wait_w(h):
            r0 = pl.ds(w * nch * 128, 64)
            pltpu.make_async_copy(ba.at[h], gs_hbm.at[r0], wa.at[h]).wait()
            pltpu.make_async_copy(bb.at[h], gd_hbm.at[r0], wb.at[h]).wait()

        start_g(0, 0)

        def body(j, carry):
            @pl.when(j > 0)
            def _():
                wait_w(1)
            start_g(j, 1)
            wait_g(0)
            start_w(j, 0)
            wait_w(0)

            @pl.when(j + 1 < nch)
            def _():
                start_g(j + 1, 0)
            wait_g(1)
            start_w(j, 1)
            return carry
        lax.fori_loop(0, nch, body, 0)
        wait_w(1)

    return k


def _make_scatter(e_pad):
    cpw = e_pad // NW
    nch = cpw // 128
    rps = N_PAD // NS
    mesh = plsc.VectorSubcoreMesh(core_axis_name="c", subcore_axis_name="s")

    @functools.partial(
        pl.kernel, mesh=mesh,
        out_type=jax.ShapeDtypeStruct((NC, N_PAD, LAT), jnp.float32),
        scratch_types=[
            pltpu.VMEM((nch, 128), jnp.int32),
            pltpu.VMEM((2, 128, LAT), jnp.float32),
            pltpu.VMEM_SHARED((N_PAD, LAT), jnp.float32),
            pltpu.SemaphoreType.DMA((2,)),
            pltpu.SemaphoreType.DMA((2,)),
            pltpu.SemaphoreType.DMA,
        ])
    def k(eu_hbm, dst_hbm, out_hbm, di_v, buf, acc, rs, ss, ds):
        c0 = lax.axis_index("c")
        s0 = lax.axis_index("s")
        w = c0 * NS + s0

        def zrow(r, c2):
            for cc in range(8):
                buf[0, r, pl.ds(cc * 16, 16)] = jnp.zeros((16,), jnp.float32)
            return c2
        lax.fori_loop(0, 128, zrow, 0, unroll=2)
        for t in range(rps // 128):
            pltpu.sync_copy(buf.at[0], acc.at[pl.ds(s0 * rps + t * 128, 128)])
        plsc.subcore_barrier()

        pltpu.sync_copy(dst_hbm.at[w], di_v)

        def start_r(cn, p):
            pltpu.async_copy(eu_hbm.at[pl.ds((w * nch + cn) * 128, 128)],
                             buf.at[p], rs.at[p])

        def wait_r(p):
            pltpu.make_async_copy(eu_hbm.at[pl.ds(w * nch * 128, 128)],
                                  buf.at[p], rs.at[p]).wait()

        def start_s(cn, p):
            pltpu.async_copy(buf.at[p], acc.at[di_v.at[cn]], ss.at[p], add=True)

        def wait_s(p):
            pltpu.make_async_copy(buf.at[p], acc.at[di_v.at[0]], ss.at[p]).wait()

        start_r(0, 0)

        def body(j2, carry):
            cn = j2 * 2

            @pl.when(j2 > 0)
            def _():
                wait_s(1)
            start_r(cn + 1, 1)
            wait_r(0)
            start_s(cn, 0)
            wait_s(0)

            @pl.when(cn + 2 < nch)
            def _():
                start_r(cn + 2, 0)
            wait_r(1)
            start_s(cn + 1, 1)
            return carry
        lax.fori_loop(0, nch // 2, body, 0)
        wait_s(1)
        plsc.subcore_barrier()

        for t in range(rps // 128):
            r0 = s0 * rps + t * 128
            pltpu.sync_copy(acc.at[pl.ds(r0, 128)], out_hbm.at[c0, pl.ds(r0, 128)])

    return k


def kernel(node_features, mesh_edge_features, world_edge_features,
           mesh_edge_index, world_edge_index, params):
    f32 = jnp.float32
    em_e = mesh_edge_features.shape[0]
    ew_e = world_edge_features.shape[0]
    msrc = jnp.pad(mesh_edge_index[0].astype(jnp.int32),
                   (0, EM_PAD - em_e)).reshape(NW, -1, 128)
    mdst = jnp.pad(mesh_edge_index[1].astype(jnp.int32), (0, EM_PAD - em_e),
                   constant_values=PAD_NODE).reshape(NW, -1, 128)
    wsrc = jnp.pad(world_edge_index[0].astype(jnp.int32),
                   (0, EW_PAD - ew_e)).reshape(NW, -1, 128)
    wdst = jnp.pad(world_edge_index[1].astype(jnp.int32), (0, EW_PAD - ew_e),
                   constant_values=PAD_NODE).reshape(NW, -1, 128)

    v = _mlp_ln(node_features.astype(f32), params['enc_node'], 2000, N_PAD)
    em = _enc_packed(mesh_edge_features.astype(f32).reshape(-1, 128),
                     params['enc_mesh'], 1000, EM_PAD)
    ew = _enc_packed(world_edge_features.astype(f32).reshape(-1, 128),
                     params['enc_world'], 1000, EW_PAD)

    gather_m = _make_gather(EM_PAD)
    gather_w = _make_gather(EW_PAD)
    scatter_m = _make_scatter(EM_PAD)
    scatter_w = _make_scatter(EW_PAD)

    n_steps = len(params['proc'])
    for si, ps in enumerate(params['proc']):
        last = si == n_steps - 1
        gs_m, gd_m = gather_m(v, msrc, mdst)
        gs_w, gd_w = gather_w(v, wsrc, wdst)
        if last:
            em_u = _edge_update_last(em, gs_m, gd_m, ps['mesh_edge'])
            ew_u = _edge_update_last(ew, gs_w, gd_w, ps['world_edge'])
        else:
            em_u, em = _edge_update(em, gs_m, gd_m, ps['mesh_edge'])
            ew_u, ew = _edge_update(ew, gs_w, gd_w, ps['world_edge'])
        am = scatter_m(em_u, mdst)
        aw = scatter_w(ew_u, wdst)
        v = _node_update(v, am, aw, ps['node'])

    dec = params['decoder']
    w2d = jnp.pad(dec['W2'], ((0, 0), (0, LAT - dec['W2'].shape[1])))
    b2d = jnp.pad(dec['b2'], (0, LAT - dec['b2'].shape[0]))
    out = _decode(v, dec['W1'], dec['b1'], w2d, b2d)
    return out[:N, :dec['W2'].shape[1]]
